# baseline (speedup 1.0000x reference)
import jax
import jax.numpy as jnp
from jax import lax
from jax.experimental import pallas as pl
from jax.experimental.pallas import tpu as pltpu
from jax.experimental.pallas import tpu_sc as plsc

N = 10000
E = 160000
F = 128
H = 128
OUT = 64
ED = 16
RELS = [("adr", "bat", 0), ("bat", "adr", 1), ("bat", "par", 2), ("par", "bat", 3), ("bat", "bat", None), ("par", "par", None), ("adr", "adr", None)]
TYPES = ["adr", "bat", "par"]
NREL = 7
NCE = 4

NC, NS, L = 2, 16, 16
NW = NC * NS
E_PAD = 163840
PER_W = E_PAD // NW
CHUNK = 64
NCH = PER_W // CHUNK
NP = 10112
STRIPE = NP // NS
KG = H // L


def _lane_sum_splat(v):
    dn = lax.GatherDimensionNumbers(
        offset_dims=(), collapsed_slice_dims=(0,), start_index_map=(0,))
    iota = lax.iota(jnp.int32, 16)
    for sh in (8, 4, 2, 1):
        idx = (iota + sh) & 15
        rot = lax.gather(v, idx[:, None], dn, slice_sizes=(1,),
                         mode=lax.GatherScatterMode.PROMISE_IN_BOUNDS)
        v = v + rot
    return v


def _layer_body(refs):
    (xl_hbm, xr_hbm, ce_hbm, srcg_hbm, dstg_hbm, dsts_hbm, aw_hbm, zr_hbm,
     acc_out, ex_out,
     src_v, dstg_v, dsts_v, a0, a1, b0, b1, c0, ex_v, aw_v, acc_s,
     si1, si2, si3, sg1, sg2, sc, ss1, ss2) = refs
    av = [a0, a1]
    bv = [b0, b1]
    cid = lax.axis_index("c")
    sid = lax.axis_index("s")
    wid = sid * NC + cid
    iota = lax.iota(jnp.int32, 16)
    zero16 = jnp.zeros((16,), jnp.float32)

    def run_relation(r, has_ce):
        base0 = r * E_PAD + wid * PER_W

        pltpu.sync_copy(zr_hbm, acc_s.at[pl.ds(sid * STRIPE, STRIPE)])
        pltpu.sync_copy(aw_hbm.at[pl.ds(r * F, F)], aw_v)
        plsc.subcore_barrier()
        awk = [aw_v[pl.ds(16 * k, 16)] for k in range(KG)]

        def issue_isd(j, q4):
            base = base0 + j * CHUNK
            pltpu.async_copy(srcg_hbm.at[pl.ds(base, CHUNK)], src_v.at[q4], si1[q4])
            pltpu.async_copy(dstg_hbm.at[pl.ds(base, CHUNK)], dstg_v.at[q4], si2[q4])

        def issue_idst(j, p):
            base = base0 + j * CHUNK
            pltpu.async_copy(dsts_hbm.at[pl.ds(base, CHUNK)], dsts_v.at[p], si3[p])

        def wait_isd(q4):
            pltpu.make_async_copy(srcg_hbm.at[pl.ds(0, CHUNK)], src_v.at[q4], si1[q4]).wait()
            pltpu.make_async_copy(dstg_hbm.at[pl.ds(0, CHUNK)], dstg_v.at[q4], si2[q4]).wait()

        def issue_g(j, q4, p):
            pltpu.async_copy(xl_hbm.at[src_v.at[q4]], av[p], sg1[p])
            pltpu.async_copy(xr_hbm.at[dstg_v.at[q4]], bv[p], sg2[p])

        def issue_ce(j):
            if has_ce:
                base = base0 + j * CHUNK
                pltpu.async_copy(ce_hbm.at[pl.ds(base, CHUNK)], c0, sc)

        def wait_g(p):
            pltpu.make_async_copy(xl_hbm.at[src_v.at[0]], av[p], sg1[p]).wait()
            pltpu.make_async_copy(xr_hbm.at[dstg_v.at[0]], bv[p], sg2[p]).wait()
            if has_ce:
                pltpu.make_async_copy(ce_hbm.at[pl.ds(0, CHUNK)], c0, sc).wait()

        def issue_s(j, p):
            base = base0 + j * CHUNK
            pltpu.async_copy(av[p], acc_s.at[dsts_v.at[p]], ss1[p], add=True)
            pltpu.async_copy(ex_v.at[p], ex_out.at[pl.ds(base, CHUNK)], ss2[p])

        def drain_s(p):
            pltpu.make_async_copy(av[p], acc_s.at[dsts_v.at[p]], ss1[p]).wait()
            pltpu.make_async_copy(ex_v.at[p], ex_out.at[pl.ds(0, CHUNK)], ss2[p]).wait()

        def wait_idst(p):
            pltpu.make_async_copy(dsts_hbm.at[pl.ds(0, CHUNK)], dsts_v.at[p], si3[p]).wait()

        def compute(p):
            a_v = av[p]
            b_v = bv[p]
            c_v = c0

            def edge_step(i, _):
                e0 = i * 2
                e1 = e0 + 1
                aks = []
                exvs = []
                for e in (e0, e1):
                    acc = zero16
                    aa = []
                    for k in range(KG):
                        ak = a_v[e, pl.ds(16 * k, 16)]
                        aa.append(ak)
                        m = ak + b_v[e, pl.ds(16 * k, 16)]
                        if has_ce:
                            m = m + c_v[e, pl.ds(16 * k, 16)]
                        lr = jnp.maximum(m, 0.2 * m)
                        acc = acc + lr * awk[k]
                    aks.append(aa)
                    exvs.append(jnp.exp(_lane_sum_splat(acc)))
                for t, e in enumerate((e0, e1)):
                    for k in range(KG):
                        a_v[e, pl.ds(16 * k, 16)] = aks[t][k] * exvs[t]
                e16 = (e0 >> 4) << 4
                win = ex_v[p, pl.ds(e16, 16)]
                win = jnp.where(iota == (e0 & 15), exvs[0], win)
                ex_v[p, pl.ds(e16, 16)] = jnp.where(iota == (e1 & 15), exvs[1], win)
                return 0

            lax.fori_loop(0, CHUNK // 2, edge_step, 0)

        def proc(j, u, first=False, pre_i2=True, pre_n1=True):
            p = u & 1
            if pre_i2:
                issue_isd(j + 2, (u + 2) & 3)
            wait_g(p)
            compute(p)
            if pre_n1:
                issue_ce(j + 1)
            if not first:
                drain_s(1 - p)
            if pre_n1:
                issue_idst(j + 1, 1 - p)
            wait_idst(p)
            issue_s(j, p)
            if pre_n1:
                wait_isd((u + 1) & 3)
                issue_g(j + 1, (u + 1) & 3, 1 - p)

        issue_isd(0, 0)
        issue_isd(1, 1)
        issue_idst(0, 0)
        issue_ce(0)
        wait_isd(0)
        issue_g(0, 0, 0)

        proc(0, 0, first=True)

        def quad_shift(t, _):
            j = 1 + t * 4
            proc(j + 0, 1)
            proc(j + 1, 2)
            proc(j + 2, 3)
            proc(j + 3, 0)
            return 0

        lax.fori_loop(0, (NCH - 4) // 4, quad_shift, 0)
        proc(NCH - 3, 1)
        proc(NCH - 2, 2, pre_i2=False)
        proc(NCH - 1, 3, pre_i2=False, pre_n1=False)
        drain_s((NCH - 1) & 1)

        plsc.subcore_barrier()
        pltpu.sync_copy(
            acc_s.at[pl.ds(sid * STRIPE, STRIPE)],
            acc_out.at[pl.ds(r * NC * NP + cid * NP + sid * STRIPE, STRIPE)])

    def rel_ce(r, _):
        run_relation(r, True)
        return 0

    def rel_nce(r, _):
        run_relation(r, False)
        return 0

    lax.fori_loop(0, NCE, rel_ce, 0)
    lax.fori_loop(NCE, NREL, rel_nce, 0)


def _make_layer_kernel():
    mesh = plsc.VectorSubcoreMesh(core_axis_name="c", subcore_axis_name="s")
    scratch = [
        pltpu.VMEM((4, CHUNK), jnp.int32),
        pltpu.VMEM((4, CHUNK), jnp.int32),
        pltpu.VMEM((2, CHUNK), jnp.int32),
        pltpu.VMEM((CHUNK, F), jnp.float32),
        pltpu.VMEM((CHUNK, F), jnp.float32),
        pltpu.VMEM((CHUNK, F), jnp.float32),
        pltpu.VMEM((CHUNK, F), jnp.float32),
        pltpu.VMEM((CHUNK, F), jnp.float32),
        pltpu.VMEM((2, CHUNK), jnp.float32),
        pltpu.VMEM((F,), jnp.float32),
        pltpu.VMEM_SHARED((NP, F), jnp.float32),
        [pltpu.SemaphoreType.DMA] * 4,
        [pltpu.SemaphoreType.DMA] * 4,
        [pltpu.SemaphoreType.DMA] * 2,
        [pltpu.SemaphoreType.DMA] * 2,
        [pltpu.SemaphoreType.DMA] * 2,
        pltpu.SemaphoreType.DMA,
        [pltpu.SemaphoreType.DMA] * 2,
        [pltpu.SemaphoreType.DMA] * 2,
    ]

    return pl.kernel(
        lambda *refs: _layer_body(refs),
        mesh=mesh,
        out_type=(
            jax.ShapeDtypeStruct((NREL * NC * NP, F), jnp.float32),
            jax.ShapeDtypeStruct((NREL * E_PAD,), jnp.float32),
        ),
        scratch_types=scratch,
    )


def _pad_e(ix, fill):
    return jnp.concatenate([ix, jnp.full((E_PAD - E,), fill, jnp.int32)])


def kernel(x_adresse, x_batiment, x_parcelle, ei_acces, ei_desservi, ei_appartient, ei_contient, ei_spat_bat, ei_spat_par, ei_spat_adr, ea_acces, ea_desservi, ea_appartient, ea_contient, Wl, Wr, att_w, bias_w, We, lin_W, lin_b):
    eis = [ei_acces, ei_desservi, ei_appartient, ei_contient, ei_spat_bat, ei_spat_par, ei_spat_adr]
    eas = [ea_acces, ea_desservi, ea_appartient, ea_contient]
    srcg = jnp.concatenate([r * N + _pad_e(eis[r][0], 0) for r in range(NREL)])
    dstgg = jnp.concatenate([r * N + _pad_e(eis[r][1], 0) for r in range(NREL)])
    dstss = jnp.concatenate([_pad_e(eis[r][1], N) for r in range(NREL)])
    zr = jnp.zeros((STRIPE, F), jnp.float32)
    ea_stack = jnp.stack(eas)
    sc_kernel = _make_layer_kernel()

    xs = {"adr": x_adresse, "bat": x_batiment, "par": x_parcelle}
    for l in range(2):
        xsrc = jnp.stack([xs[s] for s, _, _ in RELS])
        xdst = jnp.stack([xs[d] for _, d, _ in RELS])
        xl_all = jnp.einsum("rnf,rfh->rnh", xsrc, Wl[l]).reshape(NREL * N, H)
        xr_all = jnp.einsum("rnf,rfh->rnh", xdst, Wr[l]).reshape(NREL * N, H)
        ce = jnp.einsum("red,rdh->reh", ea_stack, We[l])
        ce_all = jnp.concatenate(
            [ce, jnp.zeros((NCE, E_PAD - E, H), jnp.float32)], axis=1
        ).reshape(NCE * E_PAD, H)
        aw_all = att_w[l].reshape(NREL * F)

        acc, ex = sc_kernel(xl_all, xr_all, ce_all, srcg, dstgg, dstss, aw_all, zr)
        acc = acc.reshape(NREL, NC, NP, F)
        num = (acc[:, 0] + acc[:, 1])[:, :N]
        ex = ex.reshape(NREL, E_PAD)

        new = {t: jnp.zeros((N, H), dtype=jnp.float32) for t in TYPES}
        for r, (s, d, ai) in enumerate(RELS):
            den = jax.ops.segment_sum(ex[r, :E], eis[r][1], num_segments=N)
            o = num[r] / (den + 1e-16)[:, None]
            new[d] = new[d] + o + bias_w[l, r]
        xs = {t: jax.nn.relu(v) for t, v in new.items()}
    outs = [xs[t] @ lin_W[i] + lin_b[i] for i, t in enumerate(TYPES)]
    return jnp.stack(outs)


# speedup vs baseline: 1.3031x; 1.3031x over previous
import jax
import jax.numpy as jnp
from jax import lax
from jax.experimental import pallas as pl
from jax.experimental.pallas import tpu as pltpu
from jax.experimental.pallas import tpu_sc as plsc

N = 10000
E = 160000
F = 128
H = 128
OUT = 64
ED = 16
RELS = [("adr", "bat", 0), ("bat", "adr", 1), ("bat", "par", 2), ("par", "bat", 3), ("bat", "bat", None), ("par", "par", None), ("adr", "adr", None)]
TYPES = ["adr", "bat", "par"]
NREL = 7
NCE = 4

NC, NS, L = 2, 16, 16
NW = NC * NS
E_PAD = 163840
PER_W = E_PAD // NW
CHUNK = 64
NCH = PER_W // CHUNK
NP = 10112
STRIPE = NP // NS
DR = 640
DSTRIPE = DR // NS
KG = H // L


def _lane_sum_splat(v):
    dn = lax.GatherDimensionNumbers(
        offset_dims=(), collapsed_slice_dims=(0,), start_index_map=(0,))
    iota = lax.iota(jnp.int32, 16)
    for sh in (8, 4, 2, 1):
        idx = (iota + sh) & 15
        rot = lax.gather(v, idx[:, None], dn, slice_sizes=(1,),
                         mode=lax.GatherScatterMode.PROMISE_IN_BOUNDS)
        v = v + rot
    return v


def _splat16(win, lane):
    dn = lax.GatherDimensionNumbers(
        offset_dims=(), collapsed_slice_dims=(0,), start_index_map=(0,))
    iota = lax.iota(jnp.int32, 16)
    idx = (iota & 0) + lane
    return lax.gather(win, idx[:, None], dn, slice_sizes=(1,),
                      mode=lax.GatherScatterMode.PROMISE_IN_BOUNDS)


def _layer_body(refs):
    (xl_hbm, xr_hbm, ce_hbm, srcg_hbm, dstg_hbm, dsts_hbm, dstr_hbm, aw_hbm,
     zr_hbm, acc_out, den_out,
     src_v, dstg_v, dsts_v, dstr_v, a0, a1, b0, b1, c0, aw_v, acc_s, den_s,
     si1, si2, si3, si4, sg1, sg2, sc, ss1, ss3) = refs
    av = [a0, a1]
    bv = [b0, b1]
    cid = lax.axis_index("c")
    sid = lax.axis_index("s")
    wid = sid * NC + cid
    iota = lax.iota(jnp.int32, 16)
    zero16 = jnp.zeros((16,), jnp.float32)

    def run_relation(r, has_ce):
        base0 = r * E_PAD + wid * PER_W

        pltpu.sync_copy(zr_hbm, acc_s.at[pl.ds(sid * STRIPE, STRIPE)])
        pltpu.sync_copy(zr_hbm.at[pl.ds(0, DSTRIPE)],
                        den_s.at[pl.ds(sid * DSTRIPE, DSTRIPE)])
        pltpu.sync_copy(aw_hbm.at[pl.ds(r * F, F)], aw_v)
        plsc.subcore_barrier()
        awk = [aw_v[pl.ds(16 * k, 16)] for k in range(KG)]

        def issue_isd(j, q4):
            base = base0 + j * CHUNK
            pltpu.async_copy(srcg_hbm.at[pl.ds(base, CHUNK)], src_v.at[q4], si1[q4])
            pltpu.async_copy(dstg_hbm.at[pl.ds(base, CHUNK)], dstg_v.at[q4], si2[q4])

        def issue_idst(j, p):
            base = base0 + j * CHUNK
            pltpu.async_copy(dsts_hbm.at[pl.ds(base, CHUNK)], dsts_v.at[p], si3[p])
            pltpu.async_copy(dstr_hbm.at[pl.ds(base, CHUNK)], dstr_v.at[p], si4[p])

        def wait_isd(q4):
            pltpu.make_async_copy(srcg_hbm.at[pl.ds(0, CHUNK)], src_v.at[q4], si1[q4]).wait()
            pltpu.make_async_copy(dstg_hbm.at[pl.ds(0, CHUNK)], dstg_v.at[q4], si2[q4]).wait()

        def issue_g(j, q4, p):
            pltpu.async_copy(xl_hbm.at[src_v.at[q4]], av[p], sg1[p])
            pltpu.async_copy(xr_hbm.at[dstg_v.at[q4]], bv[p], sg2[p])

        def issue_ce(j):
            if has_ce:
                base = base0 + j * CHUNK
                pltpu.async_copy(ce_hbm.at[pl.ds(base, CHUNK)], c0, sc)

        def wait_g(p):
            pltpu.make_async_copy(xl_hbm.at[src_v.at[0]], av[p], sg1[p]).wait()
            pltpu.make_async_copy(xr_hbm.at[dstg_v.at[0]], bv[p], sg2[p]).wait()
            if has_ce:
                pltpu.make_async_copy(ce_hbm.at[pl.ds(0, CHUNK)], c0, sc).wait()

        def issue_s(j, p):
            pltpu.async_copy(av[p], acc_s.at[dsts_v.at[p]], ss1[p], add=True)
            pltpu.async_copy(bv[p], den_s.at[dstr_v.at[p]], ss3[p], add=True)

        def drain_s(p):
            pltpu.make_async_copy(av[p], acc_s.at[dsts_v.at[p]], ss1[p]).wait()
            pltpu.make_async_copy(bv[p], den_s.at[dstr_v.at[p]], ss3[p]).wait()

        def wait_idst(p):
            pltpu.make_async_copy(dsts_hbm.at[pl.ds(0, CHUNK)], dsts_v.at[p], si3[p]).wait()
            pltpu.make_async_copy(dstr_hbm.at[pl.ds(0, CHUNK)], dstr_v.at[p], si4[p]).wait()

        def compute(p):
            a_v = av[p]
            b_v = bv[p]
            c_v = c0
            d_v = dsts_v.at[p]

            def edge_step(i, _):
                e0 = i * 2
                e1 = e0 + 1
                e16 = (e0 >> 4) << 4
                dwin = jnp.asarray(d_v[pl.ds(e16, 16)], jnp.float32)
                aks = []
                exvs = []
                starts = []
                for e in (e0, e1):
                    acc = zero16
                    aa = []
                    for k in range(KG):
                        ak = a_v[e, pl.ds(16 * k, 16)]
                        aa.append(ak)
                        m = ak + b_v[e, pl.ds(16 * k, 16)]
                        if has_ce:
                            m = m + c_v[e, pl.ds(16 * k, 16)]
                        lr = jnp.maximum(m, 0.2 * m)
                        acc = acc + lr * awk[k]
                    aks.append(aa)
                    exvs.append(jnp.exp(_lane_sum_splat(acc)))
                    d_splat = jnp.asarray(_splat16(dwin, e & 15), jnp.int32)
                    starts.append((d_splat & 15) * 8)
                for t, e in enumerate((e0, e1)):
                    for k in range(KG):
                        a_v[e, pl.ds(16 * k, 16)] = aks[t][k] * exvs[t]
                    start = starts[t]
                    exv = exvs[t]
                    for k in range(KG):
                        pos = iota + (16 * k)
                        sel = (pos >= start) & (pos < start + 8)
                        b_v[e, pl.ds(16 * k, 16)] = jnp.where(sel, exv, 0.0)
                return 0

            lax.fori_loop(0, CHUNK // 2, edge_step, 0)

        def proc(j, u, first=False, pre_i2=True, pre_n1=True):
            p = u & 1
            if pre_i2:
                issue_isd(j + 2, (u + 2) & 3)
            wait_g(p)
            compute(p)
            if pre_n1:
                issue_ce(j + 1)
            if not first:
                drain_s(1 - p)
            if pre_n1:
                issue_idst(j + 1, 1 - p)
            wait_idst(p)
            issue_s(j, p)
            if pre_n1:
                wait_isd((u + 1) & 3)
                issue_g(j + 1, (u + 1) & 3, 1 - p)

        issue_isd(0, 0)
        issue_isd(1, 1)
        issue_idst(0, 0)
        issue_ce(0)
        wait_isd(0)
        issue_g(0, 0, 0)

        proc(0, 0, first=True)

        def quad_shift(t, _):
            j = 1 + t * 4
            proc(j + 0, 1)
            proc(j + 1, 2)
            proc(j + 2, 3)
            proc(j + 3, 0)
            return 0

        lax.fori_loop(0, (NCH - 4) // 4, quad_shift, 0)
        proc(NCH - 3, 1)
        proc(NCH - 2, 2, pre_i2=False)
        proc(NCH - 1, 3, pre_i2=False, pre_n1=False)
        drain_s((NCH - 1) & 1)

        plsc.subcore_barrier()
        pltpu.sync_copy(
            acc_s.at[pl.ds(sid * STRIPE, STRIPE)],
            acc_out.at[pl.ds(r * NC * NP + cid * NP + sid * STRIPE, STRIPE)])
        pltpu.sync_copy(
            den_s.at[pl.ds(sid * DSTRIPE, DSTRIPE)],
            den_out.at[pl.ds(r * NC * DR + cid * DR + sid * DSTRIPE, DSTRIPE)])

    def rel_ce(r, _):
        run_relation(r, True)
        return 0

    def rel_nce(r, _):
        run_relation(r, False)
        return 0

    lax.fori_loop(0, NCE, rel_ce, 0)
    lax.fori_loop(NCE, NREL, rel_nce, 0)


def _make_layer_kernel():
    mesh = plsc.VectorSubcoreMesh(core_axis_name="c", subcore_axis_name="s")
    scratch = [
        pltpu.VMEM((4, CHUNK), jnp.int32),
        pltpu.VMEM((4, CHUNK), jnp.int32),
        pltpu.VMEM((2, CHUNK), jnp.int32),
        pltpu.VMEM((2, CHUNK), jnp.int32),
        pltpu.VMEM((CHUNK, F), jnp.float32),
        pltpu.VMEM((CHUNK, F), jnp.float32),
        pltpu.VMEM((CHUNK, F), jnp.float32),
        pltpu.VMEM((CHUNK, F), jnp.float32),
        pltpu.VMEM((CHUNK, F), jnp.float32),
        pltpu.VMEM((F,), jnp.float32),
        pltpu.VMEM_SHARED((NP, F), jnp.float32),
        pltpu.VMEM_SHARED((DR, F), jnp.float32),
        [pltpu.SemaphoreType.DMA] * 4,
        [pltpu.SemaphoreType.DMA] * 4,
        [pltpu.SemaphoreType.DMA] * 2,
        [pltpu.SemaphoreType.DMA] * 2,
        [pltpu.SemaphoreType.DMA] * 2,
        [pltpu.SemaphoreType.DMA] * 2,
        pltpu.SemaphoreType.DMA,
        [pltpu.SemaphoreType.DMA] * 2,
        [pltpu.SemaphoreType.DMA] * 2,
    ]

    return pl.kernel(
        lambda *refs: _layer_body(refs),
        mesh=mesh,
        out_type=(
            jax.ShapeDtypeStruct((NREL * NC * NP, F), jnp.float32),
            jax.ShapeDtypeStruct((NREL * NC * DR, F), jnp.float32),
        ),
        scratch_types=scratch,
    )


def _pad_e(ix, fill):
    return jnp.concatenate([ix, jnp.full((E_PAD - E,), fill, jnp.int32)])


def kernel(x_adresse, x_batiment, x_parcelle, ei_acces, ei_desservi, ei_appartient, ei_contient, ei_spat_bat, ei_spat_par, ei_spat_adr, ea_acces, ea_desservi, ea_appartient, ea_contient, Wl, Wr, att_w, bias_w, We, lin_W, lin_b):
    eis = [ei_acces, ei_desservi, ei_appartient, ei_contient, ei_spat_bat, ei_spat_par, ei_spat_adr]
    eas = [ea_acces, ea_desservi, ea_appartient, ea_contient]
    srcg = jnp.concatenate([r * N + _pad_e(eis[r][0], 0) for r in range(NREL)])
    dstgg = jnp.concatenate([r * N + _pad_e(eis[r][1], 0) for r in range(NREL)])
    dstss = jnp.concatenate([_pad_e(eis[r][1], N) for r in range(NREL)])
    dstrr = dstss >> 4
    zr = jnp.zeros((STRIPE, F), jnp.float32)
    ea_stack = jnp.stack(eas)
    sc_kernel = _make_layer_kernel()

    xs = {"adr": x_adresse, "bat": x_batiment, "par": x_parcelle}
    for l in range(2):
        xsrc = jnp.stack([xs[s] for s, _, _ in RELS])
        xdst = jnp.stack([xs[d] for _, d, _ in RELS])
        xl_all = jnp.einsum("rnf,rfh->rnh", xsrc, Wl[l]).reshape(NREL * N, H)
        xr_all = jnp.einsum("rnf,rfh->rnh", xdst, Wr[l]).reshape(NREL * N, H)
        ce = jnp.einsum("red,rdh->reh", ea_stack, We[l])
        ce_all = jnp.concatenate(
            [ce, jnp.zeros((NCE, E_PAD - E, H), jnp.float32)], axis=1
        ).reshape(NCE * E_PAD, H)
        aw_all = att_w[l].reshape(NREL * F)

        acc, dn = sc_kernel(xl_all, xr_all, ce_all, srcg, dstgg, dstss, dstrr, aw_all, zr)
        acc = acc.reshape(NREL, NC, NP, F)
        num = (acc[:, 0] + acc[:, 1])[:, :N]
        dn = dn.reshape(NREL, NC, DR, 16, 8)[:, :, :, :, 0]
        den = (dn[:, 0] + dn[:, 1]).reshape(NREL, DR * 16)[:, :N]

        new = {t: jnp.zeros((N, H), dtype=jnp.float32) for t in TYPES}
        for r, (s, d, ai) in enumerate(RELS):
            o = num[r] / (den[r] + 1e-16)[:, None]
            new[d] = new[d] + o + bias_w[l, r]
        xs = {t: jax.nn.relu(v) for t, v in new.items()}
    outs = [xs[t] @ lin_W[i] + lin_b[i] for i, t in enumerate(TYPES)]
    return jnp.stack(outs)
